# Initial kernel scaffold; baseline (speedup 1.0000x reference)
import jax
import jax.numpy as jnp
from jax.experimental import pallas as pl


def kernel(x, edge_index, wavelet_constructor):
    raise NotImplementedError("write your pallas kernel here")


# speedup vs baseline: 5.9804x; 5.9804x over previous
import functools

import jax
import jax.numpy as jnp
from jax import lax
from jax.experimental import pallas as pl
from jax.experimental.pallas import tpu as pltpu
from jax.experimental.pallas import tpu_sc as plsc

_N = 10000
_E = 320000
_C = 128

_NC = 2
_NS = 16
_NW = _NC * _NS
_EPW = _E // _NW
_BS = 128
_NB = _EPW // _BS
_REM = _EPW - _NB * _BS
_NP = 10240
_RPW = _NP // _NS
_ZR = _RPW // 5
_ND = 10240
_DPW = _ND // _NS


@functools.lru_cache(maxsize=None)
def _mesh():
    return plsc.VectorSubcoreMesh(core_axis_name="c", subcore_axis_name="s",
                                  num_cores=_NC, num_subcores=_NS)


def _zero_vmem_1d(ref, n):
    z = jnp.zeros((16,), jnp.float32)
    def body(i, _):
        ref[pl.ds(i * 16, 16)] = z
        return 0
    lax.fori_loop(0, n // 16, body, 0)


def _zero_vmem_rows(ref, rows):
    z = jnp.zeros((16,), jnp.float32)
    def body(i, _):
        for j in range(_C // 16):
            ref[i, pl.ds(j * 16, 16)] = z
        return 0
    lax.fori_loop(0, rows, body, 0)


@functools.lru_cache(maxsize=None)
def _sc_degree():
    @functools.partial(
        pl.kernel,
        out_type=jax.ShapeDtypeStruct((_NC, _ND), jnp.float32),
        mesh=_mesh(),
        scratch_types=[
            pltpu.VMEM_SHARED((_ND,), jnp.float32),
            pltpu.VMEM((_DPW,), jnp.float32),
            pltpu.VMEM((_BS,), jnp.int32),
            pltpu.VMEM((_BS,), jnp.float32),
            pltpu.VMEM((_REM,), jnp.int32),
            pltpu.SemaphoreType.DMA,
        ],
    )
    def deg_kernel(col_hbm, deg_out, sdeg, zbuf, colv, onesv, colr, sem):
        c = lax.axis_index("c")
        s = lax.axis_index("s")
        wid = s * _NC + c
        ebase = wid * _EPW

        _zero_vmem_1d(zbuf, _DPW)
        pltpu.sync_copy(zbuf, sdeg.at[pl.ds(s * _DPW, _DPW)])
        one = jnp.full((16,), 1.0, jnp.float32)
        for j in range(_BS // 16):
            onesv[pl.ds(j * 16, 16)] = one
        plsc.subcore_barrier()

        def batch(b, _):
            base = ebase + b * _BS
            pltpu.sync_copy(col_hbm.at[pl.ds(base, _BS)], colv)
            pltpu.sync_copy(onesv, sdeg.at[colv], add=True)
            return 0
        lax.fori_loop(0, _NB, batch, 0)
        pltpu.sync_copy(col_hbm.at[pl.ds(ebase + _NB * _BS, _REM)], colr)
        pltpu.sync_copy(onesv.at[pl.ds(0, _REM)], sdeg.at[colr], add=True)

        plsc.subcore_barrier()
        pltpu.sync_copy(sdeg.at[pl.ds(s * _DPW, _DPW)],
                        deg_out.at[c, pl.ds(s * _DPW, _DPW)])

    return deg_kernel


@functools.lru_cache(maxsize=None)
def _sc_step():
    @functools.partial(
        pl.kernel,
        out_type=jax.ShapeDtypeStruct((_NC, _NP, _C), jnp.float32),
        mesh=_mesh(),
        scratch_types=[
            pltpu.VMEM_SHARED((_NP, _C), jnp.float32),
            pltpu.VMEM((_ZR, _C), jnp.float32),
            pltpu.VMEM((_BS,), jnp.int32),
            pltpu.VMEM((_BS,), jnp.int32),
            pltpu.VMEM((_BS, _C), jnp.float32),
            pltpu.VMEM((_REM,), jnp.int32),
            pltpu.VMEM((_REM,), jnp.int32),
            pltpu.VMEM((_REM, _C), jnp.float32),
            pltpu.SemaphoreType.DMA,
        ],
    )
    def step_kernel(y_hbm, col_hbm, dst_hbm, acc_out,
                    sacc, zbuf, colv, dstv, stage, colr, dstr, stager, sem):
        c = lax.axis_index("c")
        s = lax.axis_index("s")
        wid = s * _NC + c
        ebase = wid * _EPW

        _zero_vmem_rows(zbuf, _ZR)
        def zcopy(t, _):
            pltpu.sync_copy(zbuf, sacc.at[pl.ds(s * _RPW + t * _ZR, _ZR)])
            return 0
        lax.fori_loop(0, 5, zcopy, 0)
        plsc.subcore_barrier()

        def batch(b, _):
            base = ebase + b * _BS
            pltpu.sync_copy(col_hbm.at[pl.ds(base, _BS)], colv)
            pltpu.sync_copy(dst_hbm.at[pl.ds(base, _BS)], dstv)
            pltpu.async_copy(y_hbm.at[colv], stage, sem).wait()
            pltpu.sync_copy(stage, sacc.at[dstv], add=True)
            return 0
        lax.fori_loop(0, _NB, batch, 0)

        rbase = ebase + _NB * _BS
        pltpu.sync_copy(col_hbm.at[pl.ds(rbase, _REM)], colr)
        pltpu.sync_copy(dst_hbm.at[pl.ds(rbase, _REM)], dstr)
        pltpu.async_copy(y_hbm.at[colr], stager, sem).wait()
        pltpu.sync_copy(stager, sacc.at[dstr], add=True)

        plsc.subcore_barrier()
        pltpu.sync_copy(sacc.at[pl.ds(s * _RPW, _RPW)],
                        acc_out.at[c, pl.ds(s * _RPW, _RPW)])

    return step_kernel


_BN = 400


def _tc_prep_body(deg_ref, x_ref, dinv_ref, y_ref):
    d = deg_ref[:, 0] + deg_ref[:, 1]
    di = jnp.where(d > 0.0, 1.0 / d, 0.0)
    dinv_ref[...] = di[:, None]
    y_ref[...] = x_ref[...] * di[:, None]


def _tc_prep(deg2, x):
    degT = jnp.transpose(deg2)
    return pl.pallas_call(
        _tc_prep_body,
        grid=(_N // _BN,),
        in_specs=[
            pl.BlockSpec((_BN, _NC), lambda i: (i, 0)),
            pl.BlockSpec((_BN, _C), lambda i: (i, 0)),
        ],
        out_specs=[
            pl.BlockSpec((_BN, 1), lambda i: (i, 0)),
            pl.BlockSpec((_BN, _C), lambda i: (i, 0)),
        ],
        out_shape=[
            jax.ShapeDtypeStruct((_N, 1), jnp.float32),
            jax.ShapeDtypeStruct((_N, _C), jnp.float32),
        ],
    )(degT, x)


def _tc_combine_body(x_ref, acc_ref, dinv_ref, xn_ref, yn_ref):
    a = acc_ref[0] + acc_ref[1]
    xn = 0.5 * (x_ref[...] + a)
    xn_ref[...] = xn
    yn_ref[...] = xn * dinv_ref[...]


def _tc_combine(x, acc, dinv):
    return pl.pallas_call(
        _tc_combine_body,
        grid=(_N // _BN,),
        in_specs=[
            pl.BlockSpec((_BN, _C), lambda i: (i, 0)),
            pl.BlockSpec((_NC, _BN, _C), lambda i: (0, i, 0)),
            pl.BlockSpec((_BN, 1), lambda i: (i, 0)),
        ],
        out_specs=[
            pl.BlockSpec((_BN, _C), lambda i: (i, 0)),
            pl.BlockSpec((_BN, _C), lambda i: (i, 0)),
        ],
        out_shape=[
            jax.ShapeDtypeStruct((_N, _C), jnp.float32),
            jax.ShapeDtypeStruct((_N, _C), jnp.float32),
        ],
    )(x, acc, dinv)


def _tc_wavelet_body(*refs):
    w_ref = refs[0]
    lev_refs = refs[1:18]
    pos_ref, neg_ref = refs[18], refs[19]
    lev = jnp.stack([r[...] for r in lev_refs])
    w = w_ref[...]
    wc = jax.lax.dot_general(w, lev.reshape(17, -1),
                             (((1,), (0,)), ((), ())),
                             preferred_element_type=jnp.float32)
    wc = wc.reshape(6, -1, _C)
    wc = jnp.transpose(wc, (1, 0, 2))
    pos_ref[...] = jax.nn.relu(wc)
    neg_ref[...] = jax.nn.relu(-wc)


def _tc_wavelet(w, levels):
    bn = 400
    pos, neg = pl.pallas_call(
        _tc_wavelet_body,
        grid=(_N // bn,),
        in_specs=[pl.BlockSpec((6, 17), lambda i: (0, 0))] +
                 [pl.BlockSpec((bn, _C), lambda i: (i, 0))] * 17,
        out_specs=[pl.BlockSpec((bn, 6, _C), lambda i: (i, 0, 0)),
                   pl.BlockSpec((bn, 6, _C), lambda i: (i, 0, 0))],
        out_shape=[jax.ShapeDtypeStruct((_N, 6, _C), jnp.float32),
                   jax.ShapeDtypeStruct((_N, 6, _C), jnp.float32)],
    )(w, *levels)
    return jnp.stack([pos, neg], axis=-1)


@jax.jit
def _run(x, edge_index, wavelet_constructor):
    dst = edge_index[0]
    col = edge_index[1]
    deg2 = _sc_degree()(col)
    dinv, y = _tc_prep(deg2, x)
    levels = [x]
    for _ in range(16):
        acc = _sc_step()(y, col, dst)
        xn, y = _tc_combine(levels[-1], acc, dinv)
        levels.append(xn)
    return _tc_wavelet(wavelet_constructor, levels)


def kernel(x, edge_index, wavelet_constructor):
    return _run(x, edge_index, wavelet_constructor)
